# Initial kernel scaffold; baseline (speedup 1.0000x reference)
import jax
import jax.numpy as jnp
from jax.experimental import pallas as pl


def kernel(x, edge_index, batch, params):
    raise NotImplementedError("write your pallas kernel here")


# speedup vs baseline: 6.9436x; 6.9436x over previous
import functools
import math

import jax
import jax.numpy as jnp
from jax import lax
from jax.experimental import pallas as pl
from jax.experimental.pallas import tpu as pltpu
from jax.experimental.pallas import tpu_sc as plsc

N = 10000
E = 320000
D = 128
H = 256
HH = 128
L = 4
G = 64
BN_BLOCK = 1000
NB = N // BN_BLOCK
BNSCALE = 1.0 / math.sqrt(1.0 + 1e-5)

NC = 2
NS = 16
CH = 80
PADN = 10112
ROWS_W = N // NS
EDGES_W_DEG = E // (NC * NS)
EDGES_W = E // NS
CHROWS = 200

_SC_CACHE = {}


def _sc_mesh():
    return plsc.VectorSubcoreMesh(core_axis_name="c", subcore_axis_name="s",
                                  num_cores=NC, num_subcores=NS)


def _relu(v):
    return jnp.maximum(v, 0.0)


def _deg_body(dst_hbm, degp_hbm, deg_s, idx_v, ones_v, zb_v):
    c = lax.axis_index("c")
    s = lax.axis_index("s")

    def fill(i, _):
        zb_v[pl.ds(i * 16, 16)] = jnp.zeros((16,), jnp.float32)
        ones_v[pl.ds((i % (CH // 16)) * 16, 16)] = jnp.ones((16,), jnp.float32)
        return 0

    lax.fori_loop(0, 40, fill, 0)
    pltpu.sync_copy(zb_v.at[pl.ds(0, 632)], deg_s.at[pl.ds(s * 632, 632)])
    plsc.subcore_barrier()

    base = (c * NS + s) * EDGES_W_DEG

    def body(i, _):
        pltpu.sync_copy(dst_hbm.at[pl.ds(base + i * CH, CH)], idx_v)
        pltpu.sync_copy(ones_v, deg_s.at[idx_v], add=True)
        return 0

    lax.fori_loop(0, EDGES_W_DEG // CH, body, 0)
    plsc.subcore_barrier()
    pltpu.sync_copy(deg_s.at[pl.ds(s * 632, 632)], zb_v.at[pl.ds(0, 632)])
    pltpu.sync_copy(zb_v.at[pl.ds(0, 632)],
                    degp_hbm.at[pl.ds(c * PADN + s * 632, 632)])


def _sc_degree(dst):
    if 'deg' not in _SC_CACHE:
        _SC_CACHE['deg'] = pl.kernel(
            _deg_body,
            out_type=jax.ShapeDtypeStruct((NC * PADN,), jnp.float32),
            mesh=_sc_mesh(),
            scratch_types=[
                pltpu.VMEM_SHARED((PADN,), jnp.float32),
                pltpu.VMEM((CH,), jnp.int32),
                pltpu.VMEM((CH,), jnp.float32),
                pltpu.VMEM((640,), jnp.float32),
            ],
        )
    return _SC_CACHE['deg'](dst)


def _scat_body(g2_hbm, src2_hbm, dst_hbm, acc_hbm, acc_s, sidx_v, didx_v,
               rows_v, bounce_v, sem):
    c = lax.axis_index("c")
    s = lax.axis_index("s")
    def init(j, _):
        k = j * NS + s

        @pl.when(k < N // CHROWS)
        def _():
            r = k * CHROWS
            pltpu.sync_copy(g2_hbm.at[pl.ds(c * N + r, CHROWS)], bounce_v)
            pltpu.sync_copy(bounce_v, acc_s.at[pl.ds(r, CHROWS)])

        return 0

    lax.fori_loop(0, (N // CHROWS + NS - 1) // NS, init, 0)
    plsc.subcore_barrier()

    ebase = s * EDGES_W

    def body(i, _):
        off = ebase + i * CH
        pltpu.sync_copy(src2_hbm.at[pl.ds(c * E + off, CH)], sidx_v)
        pltpu.sync_copy(dst_hbm.at[pl.ds(off, CH)], didx_v)
        pltpu.async_copy(g2_hbm.at[sidx_v], rows_v, sem).wait()
        pltpu.sync_copy(rows_v, acc_s.at[didx_v], add=True)
        return 0

    lax.fori_loop(0, EDGES_W // CH, body, 0)
    plsc.subcore_barrier()

    def wb(j, _):
        k = j * NS + s

        @pl.when(k < N // CHROWS)
        def _():
            r = k * CHROWS
            pltpu.sync_copy(acc_s.at[pl.ds(r, CHROWS)], bounce_v)
            pltpu.sync_copy(bounce_v, acc_hbm.at[pl.ds(c * N + r, CHROWS)])

        return 0

    lax.fori_loop(0, (N // CHROWS + NS - 1) // NS, wb, 0)


def _edge_scatter(g, src2, dst):
    if 'scat' not in _SC_CACHE:
        _SC_CACHE['scat'] = pl.kernel(
            _scat_body,
            out_type=jax.ShapeDtypeStruct((NC * N, HH), jnp.float32),
            mesh=_sc_mesh(),
            scratch_types=[
                pltpu.VMEM_SHARED((N, HH), jnp.float32),
                pltpu.VMEM((CH,), jnp.int32),
                pltpu.VMEM((CH,), jnp.int32),
                pltpu.VMEM((CH, HH), jnp.float32),
                pltpu.VMEM((CHROWS, HH), jnp.float32),
                pltpu.SemaphoreType.DMA,
            ],
        )
    g2 = g.reshape(NC * N, HH)
    acc2 = _SC_CACHE['scat'](g2, src2, dst)
    return acc2.reshape(NC, N, HH)


def _dinv_of(degp_ref):
    deg = degp_ref[0] + degp_ref[1] + 1.0
    return lax.rsqrt(jnp.maximum(deg, 1.0))


def _mm0_body(x_ref, w_ref, degp_ref, o_ref):
    dinv = _dinv_of(degp_ref)
    h = jnp.dot(x_ref[...], w_ref[...], preferred_element_type=jnp.float32)
    o_ref[0] = h * dinv


def _mm0(x, w, degp3):
    return pl.pallas_call(
        _mm0_body,
        grid=(NB, 2),
        in_specs=[
            pl.BlockSpec((BN_BLOCK, D), lambda i, c: (i, 0)),
            pl.BlockSpec((D, HH), lambda i, c: (0, c)),
            pl.BlockSpec((2, BN_BLOCK, 1), lambda i, c: (0, i, 0)),
        ],
        out_specs=pl.BlockSpec((1, BN_BLOCK, HH), lambda i, c: (c, i, 0)),
        out_shape=jax.ShapeDtypeStruct((2, N, HH), jnp.float32),
    )(x, w, degp3)


def _mm_body(s0_ref, s1_ref, w0_ref, w1_ref, degp_ref, o_ref):
    dinv = _dinv_of(degp_ref)
    h = jnp.dot(s0_ref[0], w0_ref[0], preferred_element_type=jnp.float32)
    h = h + jnp.dot(s1_ref[0], w1_ref[0], preferred_element_type=jnp.float32)
    o_ref[0] = h * dinv


def _mm(s, w2, degp3):
    return pl.pallas_call(
        _mm_body,
        grid=(NB, 2),
        in_specs=[
            pl.BlockSpec((1, BN_BLOCK, HH), lambda i, c: (0, i, 0)),
            pl.BlockSpec((1, BN_BLOCK, HH), lambda i, c: (1, i, 0)),
            pl.BlockSpec((1, HH, HH), lambda i, c: (0, 0, c)),
            pl.BlockSpec((1, HH, HH), lambda i, c: (1, 0, c)),
            pl.BlockSpec((2, BN_BLOCK, 1), lambda i, c: (0, i, 0)),
        ],
        out_specs=pl.BlockSpec((1, BN_BLOCK, HH), lambda i, c: (c, i, 0)),
        out_shape=jax.ShapeDtypeStruct((2, N, HH), jnp.float32),
    )(s, s, w2, w2, degp3)


def _combine_body(acc_ref, degp_ref, b_ref, bng_ref, bnb_ref, o_ref):
    dinv = _dinv_of(degp_ref)
    t = acc_ref[0] * dinv + b_ref[0]
    t = t * (BNSCALE * bng_ref[0]) + bnb_ref[0]
    o_ref[0] = _relu(t)


def _combine_res_body(acc_ref, degp_ref, b_ref, bng_ref, bnb_ref, sprev_ref,
                      o_ref):
    dinv = _dinv_of(degp_ref)
    t = acc_ref[0] * dinv + b_ref[0]
    t = t * (BNSCALE * bng_ref[0]) + bnb_ref[0]
    o_ref[0] = _relu(t) + sprev_ref[0]


def _combine(acc, degp3, b2, bng2, bnb2, sprev=None):
    specs = [
        pl.BlockSpec((1, BN_BLOCK, HH), lambda i, c: (c, i, 0)),
        pl.BlockSpec((2, BN_BLOCK, 1), lambda i, c: (0, i, 0)),
        pl.BlockSpec((1, 1, HH), lambda i, c: (c, 0, 0)),
        pl.BlockSpec((1, 1, HH), lambda i, c: (c, 0, 0)),
        pl.BlockSpec((1, 1, HH), lambda i, c: (c, 0, 0)),
    ]
    args = [acc, degp3, b2, bng2, bnb2]
    body = _combine_body
    if sprev is not None:
        specs.append(pl.BlockSpec((1, BN_BLOCK, HH), lambda i, c: (c, i, 0)))
        args.append(sprev)
        body = _combine_res_body
    return pl.pallas_call(
        body,
        grid=(NB, 2),
        in_specs=specs,
        out_specs=pl.BlockSpec((1, BN_BLOCK, HH), lambda i, c: (c, i, 0)),
        out_shape=jax.ShapeDtypeStruct((2, N, HH), jnp.float32),
    )(*args)


def _pool_body(s0_ref, s1_ref, x_ref, b_ref, o_gx_ref, o_x_ref, o_cnt_ref):
    i = pl.program_id(0)

    @pl.when(i == 0)
    def _():
        o_gx_ref[...] = jnp.zeros_like(o_gx_ref)
        o_x_ref[...] = jnp.zeros_like(o_x_ref)
        o_cnt_ref[...] = jnp.zeros_like(o_cnt_ref)

    b = b_ref[0]
    gid = lax.broadcasted_iota(jnp.int32, (G, BN_BLOCK), 0)
    m = (gid == b).astype(jnp.float32)
    o_gx_ref[0] += jnp.dot(m, s0_ref[0], preferred_element_type=jnp.float32)
    o_gx_ref[1] += jnp.dot(m, s1_ref[0], preferred_element_type=jnp.float32)
    o_x_ref[...] += jnp.dot(m, x_ref[...], preferred_element_type=jnp.float32)
    o_cnt_ref[...] += jnp.broadcast_to(
        jnp.sum(m, axis=1, keepdims=True), (G, HH))


def _pool(s, x, batch3):
    return pl.pallas_call(
        _pool_body,
        grid=(NB,),
        in_specs=[
            pl.BlockSpec((1, BN_BLOCK, HH), lambda i: (0, i, 0)),
            pl.BlockSpec((1, BN_BLOCK, HH), lambda i: (1, i, 0)),
            pl.BlockSpec((BN_BLOCK, D), lambda i: (i, 0)),
            pl.BlockSpec((1, 1, BN_BLOCK), lambda i: (i, 0, 0)),
        ],
        out_specs=[
            pl.BlockSpec((2, G, HH), lambda i: (0, 0, 0)),
            pl.BlockSpec((G, D), lambda i: (0, 0)),
            pl.BlockSpec((G, HH), lambda i: (0, 0)),
        ],
        out_shape=[
            jax.ShapeDtypeStruct((2, G, HH), jnp.float32),
            jax.ShapeDtypeStruct((G, D), jnp.float32),
            jax.ShapeDtypeStruct((G, HH), jnp.float32),
        ],
    )(s, s, x, batch3)


def _head_body(sgx_ref, sx_ref, cnt_ref, dW1_ref, db1_ref, dg1_ref, dbb1_ref,
               dW2_ref, db2_ref, dg2_ref, dbb2_ref, fW1_ref, fb1_ref, fg1_ref,
               fbb1_ref, fW2_ref, fb2_ref, fg2_ref, fbb2_ref, fW3_ref,
               fb3_ref, o_ref):
    cnt = jnp.maximum(cnt_ref[...], 1.0)
    gr0 = sgx_ref[0] / cnt
    gr1 = sgx_ref[1] / cnt
    nr = sx_ref[...] / cnt[:, :D]

    def dot(a, b):
        return jnp.dot(a, b, preferred_element_type=jnp.float32)

    d1 = _relu((dot(nr, dW1_ref[...]) + db1_ref[...]) * (BNSCALE * dg1_ref[...])
               + dbb1_ref[...])
    d2 = _relu((dot(d1, dW2_ref[...]) + db2_ref[...]) * (BNSCALE * dg2_ref[...])
               + dbb2_ref[...])
    y1 = (dot(gr0, fW1_ref[0:HH]) + dot(gr1, fW1_ref[HH:2 * HH])
          + dot(d2, fW1_ref[2 * HH:4 * HH]) + fb1_ref[...])
    y1 = _relu(y1 * (BNSCALE * fg1_ref[...]) + fbb1_ref[...])
    y2 = _relu((dot(y1, fW2_ref[...]) + fb2_ref[...]) * (BNSCALE * fg2_ref[...])
               + fbb2_ref[...])
    o_ref[...] = dot(y2, fW3_ref[...]) + fb3_ref[...]


def _head(sgx, sx, cnt, p):
    r = lambda v: v.reshape(1, -1)
    args = [sgx, sx, cnt,
            p['d_W1'], r(p['d_b1']), r(p['d_bn_g1']), r(p['d_bn_b1']),
            p['d_W2'], r(p['d_b2']), r(p['d_bn_g2']), r(p['d_bn_b2']),
            p['f_W1'], r(p['f_b1']), r(p['f_bn_g1']), r(p['f_bn_b1']),
            p['f_W2'], r(p['f_b2']), r(p['f_bn_g2']), r(p['f_bn_b2']),
            p['f_W3'], r(p['f_b3'])]
    return pl.pallas_call(
        _head_body,
        out_shape=jax.ShapeDtypeStruct((G, 1), jnp.float32),
    )(*args)


def kernel(x, edge_index, batch, params):
    src = edge_index[0]
    dst = edge_index[1]
    src2 = jnp.concatenate([src, src + N])
    degp = _sc_degree(dst).reshape(NC, PADN)
    degp3 = degp[:, :N].reshape(2, N, 1)
    batch3 = batch.reshape(NB, 1, BN_BLOCK)

    g = _mm0(x, params['gcn_W_0'], degp3)
    acc = _edge_scatter(g, src2, dst)
    s = _combine(acc, degp3,
                 params['gcn_b_0'].reshape(2, 1, HH),
                 params['gcn_bn_g_0'].reshape(2, 1, HH),
                 params['gcn_bn_b_0'].reshape(2, 1, HH))
    for i in range(1, L):
        w2 = params[f'gcn_W_{i}'].reshape(2, HH, H)
        g = _mm(s, w2, degp3)
        acc = _edge_scatter(g, src2, dst)
        s = _combine(acc, degp3,
                     params[f'gcn_b_{i}'].reshape(2, 1, HH),
                     params[f'gcn_bn_g_{i}'].reshape(2, 1, HH),
                     params[f'gcn_bn_b_{i}'].reshape(2, 1, HH),
                     sprev=s)

    sgx, sx, cnt = _pool(s, x, batch3)
    return _head(sgx, sx, cnt, params)


# speedup vs baseline: 15.8221x; 2.2786x over previous
import functools
import math

import jax
import jax.numpy as jnp
from jax import lax
from jax.experimental import pallas as pl
from jax.experimental.pallas import tpu as pltpu
from jax.experimental.pallas import tpu_sc as plsc

N = 10000
E = 320000
D = 128
H = 256
HH = 128
L = 4
G = 64
BN_BLOCK = 1000
NB = N // BN_BLOCK
BNSCALE = 1.0 / math.sqrt(1.0 + 1e-5)

NC = 2
NS = 16
CH = 80
PADN = 10112
ROWS_W = N // NS
EDGES_W_DEG = E // (NC * NS)
EDGES_W = E // NS
CHROWS = 200

_SC_CACHE = {}


def _sc_mesh():
    return plsc.VectorSubcoreMesh(core_axis_name="c", subcore_axis_name="s",
                                  num_cores=NC, num_subcores=NS)


def _relu(v):
    return jnp.maximum(v, 0.0)


def _deg_body(dst_hbm, degp_hbm, deg_s, idx_v, ones_v, zb_v):
    c = lax.axis_index("c")
    s = lax.axis_index("s")

    def fill(i, _):
        zb_v[pl.ds(i * 16, 16)] = jnp.zeros((16,), jnp.float32)
        ones_v[pl.ds((i % (CH // 16)) * 16, 16)] = jnp.ones((16,), jnp.float32)
        return 0

    lax.fori_loop(0, 40, fill, 0)
    pltpu.sync_copy(zb_v.at[pl.ds(0, 632)], deg_s.at[pl.ds(s * 632, 632)])
    plsc.subcore_barrier()

    base = (c * NS + s) * EDGES_W_DEG

    def body(i, _):
        pltpu.sync_copy(dst_hbm.at[pl.ds(base + i * CH, CH)], idx_v)
        pltpu.sync_copy(ones_v, deg_s.at[idx_v], add=True)
        return 0

    lax.fori_loop(0, EDGES_W_DEG // CH, body, 0)
    plsc.subcore_barrier()
    pltpu.sync_copy(deg_s.at[pl.ds(s * 632, 632)], zb_v.at[pl.ds(0, 632)])
    pltpu.sync_copy(zb_v.at[pl.ds(0, 632)],
                    degp_hbm.at[pl.ds(c * PADN + s * 632, 632)])


def _sc_degree(dst):
    if 'deg' not in _SC_CACHE:
        _SC_CACHE['deg'] = pl.kernel(
            _deg_body,
            out_type=jax.ShapeDtypeStruct((NC * PADN,), jnp.float32),
            mesh=_sc_mesh(),
            scratch_types=[
                pltpu.VMEM_SHARED((PADN,), jnp.float32),
                pltpu.VMEM((CH,), jnp.int32),
                pltpu.VMEM((CH,), jnp.float32),
                pltpu.VMEM((640,), jnp.float32),
            ],
        )
    return _SC_CACHE['deg'](dst)


NSET = 10
PREF = 3


def _scat_body(g2_hbm, src2_hbm, dst_hbm, acc_hbm, acc_s, sidx_v, didx_v,
               rows_v, gsem, ssem, isem):
    c = lax.axis_index("c")
    s = lax.axis_index("s")
    nchunks = EDGES_W // CH

    bounce = rows_v.at[0]

    def init(j, _):
        k = j * NS + s

        @pl.when(k < N // CH)
        def _():
            r = k * CH
            pltpu.sync_copy(g2_hbm.at[pl.ds(c * N + r, CH)], bounce)
            pltpu.sync_copy(bounce, acc_s.at[pl.ds(r, CH)])

        return 0

    lax.fori_loop(0, (N // CH + NS - 1) // NS, init, 0)
    plsc.subcore_barrier()

    ebase = s * EDGES_W

    def idx_load(chunk, q):
        off = ebase + chunk * CH
        pltpu.async_copy(src2_hbm.at[pl.ds(c * E + off, CH)],
                         sidx_v.at[q], isem.at[q])
        pltpu.async_copy(dst_hbm.at[pl.ds(off, CH)],
                         didx_v.at[q], isem.at[q])

    def idx_wait(chunk, q):
        off = ebase + chunk * CH
        pltpu.make_async_copy(src2_hbm.at[pl.ds(c * E + off, CH)],
                              sidx_v.at[q], isem.at[q]).wait()
        pltpu.make_async_copy(dst_hbm.at[pl.ds(off, CH)],
                              didx_v.at[q], isem.at[q]).wait()

    def gather_start(q, b):
        pltpu.async_copy(g2_hbm.at[sidx_v.at[q]], rows_v.at[b], gsem.at[b])

    def gather_wait(q, b):
        pltpu.make_async_copy(g2_hbm.at[sidx_v.at[q]], rows_v.at[b],
                              gsem.at[b]).wait()

    def scat_start(q, b):
        pltpu.async_copy(rows_v.at[b], acc_s.at[didx_v.at[q]], ssem.at[b],
                         add=True)

    def scat_wait(q, b):
        pltpu.make_async_copy(rows_v.at[b], acc_s.at[didx_v.at[q]],
                              ssem.at[b]).wait()

    for q in range(PREF):
        idx_load(q, q)

    def ring(t, _):
        for u in range(NSET):
            j = t * NSET + u
            b = u % 2
            if u >= 2:
                scat_wait((u - 2) % NSET, b)
            else:
                @pl.when(j >= 2)
                def _():
                    scat_wait((u - 2) % NSET, b)
            idx_wait(j, u)
            gather_start(u, b)
            if u >= 1:
                gather_wait(u - 1, 1 - b)
                scat_start(u - 1, 1 - b)
            else:
                @pl.when(j >= 1)
                def _():
                    gather_wait((u - 1) % NSET, 1 - b)
                    scat_start((u - 1) % NSET, 1 - b)
            if u < NSET - PREF:
                idx_load(j + PREF, u + PREF)
            else:
                @pl.when(j + PREF < nchunks)
                def _():
                    idx_load(j + PREF, (u + PREF) % NSET)

        return 0

    lax.fori_loop(0, nchunks // NSET, ring, 0)

    last = nchunks - 1
    gather_wait((last) % NSET, last % 2)
    scat_start((last) % NSET, last % 2)
    scat_wait((last - 1) % NSET, (last - 1) % 2)
    scat_wait((last) % NSET, last % 2)
    plsc.subcore_barrier()

    def wb(j, _):
        k = j * NS + s

        @pl.when(k < N // CH)
        def _():
            r = k * CH
            pltpu.sync_copy(acc_s.at[pl.ds(r, CH)], bounce)
            pltpu.sync_copy(bounce, acc_hbm.at[pl.ds(c * N + r, CH)])

        return 0

    lax.fori_loop(0, (N // CH + NS - 1) // NS, wb, 0)


def _edge_scatter(g, src2, dst):
    if 'scat' not in _SC_CACHE:
        _SC_CACHE['scat'] = pl.kernel(
            _scat_body,
            out_type=jax.ShapeDtypeStruct((NC * N, HH), jnp.float32),
            mesh=_sc_mesh(),
            scratch_types=[
                pltpu.VMEM_SHARED((N, HH), jnp.float32),
                pltpu.VMEM((NSET, CH), jnp.int32),
                pltpu.VMEM((NSET, CH), jnp.int32),
                pltpu.VMEM((2, CH, HH), jnp.float32),
                pltpu.SemaphoreType.DMA((2,)),
                pltpu.SemaphoreType.DMA((2,)),
                pltpu.SemaphoreType.DMA((NSET,)),
            ],
        )
    g2 = g.reshape(NC * N, HH)
    acc2 = _SC_CACHE['scat'](g2, src2, dst)
    return acc2.reshape(NC, N, HH)


def _dinv_of(degp_ref):
    deg = degp_ref[0] + degp_ref[1] + 1.0
    return lax.rsqrt(jnp.maximum(deg, 1.0))


def _mm0_body(x_ref, w_ref, degp_ref, o_ref):
    dinv = _dinv_of(degp_ref)
    h = jnp.dot(x_ref[...], w_ref[...], preferred_element_type=jnp.float32)
    o_ref[0] = h * dinv


def _mm0(x, w, degp3):
    return pl.pallas_call(
        _mm0_body,
        grid=(NB, 2),
        in_specs=[
            pl.BlockSpec((BN_BLOCK, D), lambda i, c: (i, 0)),
            pl.BlockSpec((D, HH), lambda i, c: (0, c)),
            pl.BlockSpec((2, BN_BLOCK, 1), lambda i, c: (0, i, 0)),
        ],
        out_specs=pl.BlockSpec((1, BN_BLOCK, HH), lambda i, c: (c, i, 0)),
        out_shape=jax.ShapeDtypeStruct((2, N, HH), jnp.float32),
    )(x, w, degp3)


def _mm_body(s0_ref, s1_ref, w0_ref, w1_ref, degp_ref, o_ref):
    dinv = _dinv_of(degp_ref)
    h = jnp.dot(s0_ref[0], w0_ref[0], preferred_element_type=jnp.float32)
    h = h + jnp.dot(s1_ref[0], w1_ref[0], preferred_element_type=jnp.float32)
    o_ref[0] = h * dinv


def _mm(s, w2, degp3):
    return pl.pallas_call(
        _mm_body,
        grid=(NB, 2),
        in_specs=[
            pl.BlockSpec((1, BN_BLOCK, HH), lambda i, c: (0, i, 0)),
            pl.BlockSpec((1, BN_BLOCK, HH), lambda i, c: (1, i, 0)),
            pl.BlockSpec((1, HH, HH), lambda i, c: (0, 0, c)),
            pl.BlockSpec((1, HH, HH), lambda i, c: (1, 0, c)),
            pl.BlockSpec((2, BN_BLOCK, 1), lambda i, c: (0, i, 0)),
        ],
        out_specs=pl.BlockSpec((1, BN_BLOCK, HH), lambda i, c: (c, i, 0)),
        out_shape=jax.ShapeDtypeStruct((2, N, HH), jnp.float32),
    )(s, s, w2, w2, degp3)


def _combine_body(acc_ref, degp_ref, b_ref, bng_ref, bnb_ref, o_ref):
    dinv = _dinv_of(degp_ref)
    t = acc_ref[0] * dinv + b_ref[0]
    t = t * (BNSCALE * bng_ref[0]) + bnb_ref[0]
    o_ref[0] = _relu(t)


def _combine_res_body(acc_ref, degp_ref, b_ref, bng_ref, bnb_ref, sprev_ref,
                      o_ref):
    dinv = _dinv_of(degp_ref)
    t = acc_ref[0] * dinv + b_ref[0]
    t = t * (BNSCALE * bng_ref[0]) + bnb_ref[0]
    o_ref[0] = _relu(t) + sprev_ref[0]


def _combine(acc, degp3, b2, bng2, bnb2, sprev=None):
    specs = [
        pl.BlockSpec((1, BN_BLOCK, HH), lambda i, c: (c, i, 0)),
        pl.BlockSpec((2, BN_BLOCK, 1), lambda i, c: (0, i, 0)),
        pl.BlockSpec((1, 1, HH), lambda i, c: (c, 0, 0)),
        pl.BlockSpec((1, 1, HH), lambda i, c: (c, 0, 0)),
        pl.BlockSpec((1, 1, HH), lambda i, c: (c, 0, 0)),
    ]
    args = [acc, degp3, b2, bng2, bnb2]
    body = _combine_body
    if sprev is not None:
        specs.append(pl.BlockSpec((1, BN_BLOCK, HH), lambda i, c: (c, i, 0)))
        args.append(sprev)
        body = _combine_res_body
    return pl.pallas_call(
        body,
        grid=(NB, 2),
        in_specs=specs,
        out_specs=pl.BlockSpec((1, BN_BLOCK, HH), lambda i, c: (c, i, 0)),
        out_shape=jax.ShapeDtypeStruct((2, N, HH), jnp.float32),
    )(*args)


def _pool_body(s0_ref, s1_ref, x_ref, b_ref, o_gx_ref, o_x_ref, o_cnt_ref):
    i = pl.program_id(0)

    @pl.when(i == 0)
    def _():
        o_gx_ref[...] = jnp.zeros_like(o_gx_ref)
        o_x_ref[...] = jnp.zeros_like(o_x_ref)
        o_cnt_ref[...] = jnp.zeros_like(o_cnt_ref)

    b = b_ref[0]
    gid = lax.broadcasted_iota(jnp.int32, (G, BN_BLOCK), 0)
    m = (gid == b).astype(jnp.float32)
    o_gx_ref[0] += jnp.dot(m, s0_ref[0], preferred_element_type=jnp.float32)
    o_gx_ref[1] += jnp.dot(m, s1_ref[0], preferred_element_type=jnp.float32)
    o_x_ref[...] += jnp.dot(m, x_ref[...], preferred_element_type=jnp.float32)
    o_cnt_ref[...] += jnp.broadcast_to(
        jnp.sum(m, axis=1, keepdims=True), (G, HH))


def _pool(s, x, batch3):
    return pl.pallas_call(
        _pool_body,
        grid=(NB,),
        in_specs=[
            pl.BlockSpec((1, BN_BLOCK, HH), lambda i: (0, i, 0)),
            pl.BlockSpec((1, BN_BLOCK, HH), lambda i: (1, i, 0)),
            pl.BlockSpec((BN_BLOCK, D), lambda i: (i, 0)),
            pl.BlockSpec((1, 1, BN_BLOCK), lambda i: (i, 0, 0)),
        ],
        out_specs=[
            pl.BlockSpec((2, G, HH), lambda i: (0, 0, 0)),
            pl.BlockSpec((G, D), lambda i: (0, 0)),
            pl.BlockSpec((G, HH), lambda i: (0, 0)),
        ],
        out_shape=[
            jax.ShapeDtypeStruct((2, G, HH), jnp.float32),
            jax.ShapeDtypeStruct((G, D), jnp.float32),
            jax.ShapeDtypeStruct((G, HH), jnp.float32),
        ],
    )(s, s, x, batch3)


def _head_body(sgx_ref, sx_ref, cnt_ref, dW1_ref, db1_ref, dg1_ref, dbb1_ref,
               dW2_ref, db2_ref, dg2_ref, dbb2_ref, fW1_ref, fb1_ref, fg1_ref,
               fbb1_ref, fW2_ref, fb2_ref, fg2_ref, fbb2_ref, fW3_ref,
               fb3_ref, o_ref):
    cnt = jnp.maximum(cnt_ref[...], 1.0)
    gr0 = sgx_ref[0] / cnt
    gr1 = sgx_ref[1] / cnt
    nr = sx_ref[...] / cnt[:, :D]

    def dot(a, b):
        return jnp.dot(a, b, preferred_element_type=jnp.float32)

    d1 = _relu((dot(nr, dW1_ref[...]) + db1_ref[...]) * (BNSCALE * dg1_ref[...])
               + dbb1_ref[...])
    d2 = _relu((dot(d1, dW2_ref[...]) + db2_ref[...]) * (BNSCALE * dg2_ref[...])
               + dbb2_ref[...])
    y1 = (dot(gr0, fW1_ref[0:HH]) + dot(gr1, fW1_ref[HH:2 * HH])
          + dot(d2, fW1_ref[2 * HH:4 * HH]) + fb1_ref[...])
    y1 = _relu(y1 * (BNSCALE * fg1_ref[...]) + fbb1_ref[...])
    y2 = _relu((dot(y1, fW2_ref[...]) + fb2_ref[...]) * (BNSCALE * fg2_ref[...])
               + fbb2_ref[...])
    o_ref[...] = dot(y2, fW3_ref[...]) + fb3_ref[...]


def _head(sgx, sx, cnt, p):
    r = lambda v: v.reshape(1, -1)
    args = [sgx, sx, cnt,
            p['d_W1'], r(p['d_b1']), r(p['d_bn_g1']), r(p['d_bn_b1']),
            p['d_W2'], r(p['d_b2']), r(p['d_bn_g2']), r(p['d_bn_b2']),
            p['f_W1'], r(p['f_b1']), r(p['f_bn_g1']), r(p['f_bn_b1']),
            p['f_W2'], r(p['f_b2']), r(p['f_bn_g2']), r(p['f_bn_b2']),
            p['f_W3'], r(p['f_b3'])]
    return pl.pallas_call(
        _head_body,
        out_shape=jax.ShapeDtypeStruct((G, 1), jnp.float32),
    )(*args)


def kernel(x, edge_index, batch, params):
    src = edge_index[0]
    dst = edge_index[1]
    src2 = jnp.concatenate([src, src + N])
    degp = _sc_degree(dst).reshape(NC, PADN)
    degp3 = degp[:, :N].reshape(2, N, 1)
    batch3 = batch.reshape(NB, 1, BN_BLOCK)

    g = _mm0(x, params['gcn_W_0'], degp3)
    acc = _edge_scatter(g, src2, dst)
    s = _combine(acc, degp3,
                 params['gcn_b_0'].reshape(2, 1, HH),
                 params['gcn_bn_g_0'].reshape(2, 1, HH),
                 params['gcn_bn_b_0'].reshape(2, 1, HH))
    for i in range(1, L):
        w2 = params[f'gcn_W_{i}'].reshape(2, HH, H)
        g = _mm(s, w2, degp3)
        acc = _edge_scatter(g, src2, dst)
        s = _combine(acc, degp3,
                     params[f'gcn_b_{i}'].reshape(2, 1, HH),
                     params[f'gcn_bn_g_{i}'].reshape(2, 1, HH),
                     params[f'gcn_bn_b_{i}'].reshape(2, 1, HH),
                     sprev=s)

    sgx, sx, cnt = _pool(s, x, batch3)
    return _head(sgx, sx, cnt, params)


# speedup vs baseline: 17.9122x; 1.1321x over previous
import functools
import math

import jax
import jax.numpy as jnp
from jax import lax
from jax.experimental import pallas as pl
from jax.experimental.pallas import tpu as pltpu
from jax.experimental.pallas import tpu_sc as plsc

N = 10000
E = 320000
D = 128
H = 256
HH = 128
L = 4
G = 64
BN_BLOCK = 1000
NB = N // BN_BLOCK
BNSCALE = 1.0 / math.sqrt(1.0 + 1e-5)

NC = 2
NS = 16
CH = 80
PADN = 10112
ROWS_W = N // NS
EDGES_W_DEG = E // (NC * NS)
EDGES_W = E // NS
CHROWS = 200

_SC_CACHE = {}


def _sc_mesh():
    return plsc.VectorSubcoreMesh(core_axis_name="c", subcore_axis_name="s",
                                  num_cores=NC, num_subcores=NS)


def _relu(v):
    return jnp.maximum(v, 0.0)


def _deg_body(dst_hbm, degp_hbm, deg_s, idx_v, ones_v, zb_v):
    c = lax.axis_index("c")
    s = lax.axis_index("s")

    def fill(i, _):
        zb_v[pl.ds(i * 16, 16)] = jnp.zeros((16,), jnp.float32)
        ones_v[pl.ds((i % (CH // 16)) * 16, 16)] = jnp.ones((16,), jnp.float32)
        return 0

    lax.fori_loop(0, 40, fill, 0)
    pltpu.sync_copy(zb_v.at[pl.ds(0, 632)], deg_s.at[pl.ds(s * 632, 632)])
    plsc.subcore_barrier()

    base = (c * NS + s) * EDGES_W_DEG

    def body(i, _):
        pltpu.sync_copy(dst_hbm.at[pl.ds(base + i * CH, CH)], idx_v)
        pltpu.sync_copy(ones_v, deg_s.at[idx_v], add=True)
        return 0

    lax.fori_loop(0, EDGES_W_DEG // CH, body, 0)
    plsc.subcore_barrier()
    pltpu.sync_copy(deg_s.at[pl.ds(s * 632, 632)], zb_v.at[pl.ds(0, 632)])
    pltpu.sync_copy(zb_v.at[pl.ds(0, 632)],
                    degp_hbm.at[pl.ds(c * PADN + s * 632, 632)])


def _sc_degree(dst):
    if 'deg' not in _SC_CACHE:
        _SC_CACHE['deg'] = pl.kernel(
            _deg_body,
            out_type=jax.ShapeDtypeStruct((NC * PADN,), jnp.float32),
            mesh=_sc_mesh(),
            scratch_types=[
                pltpu.VMEM_SHARED((PADN,), jnp.float32),
                pltpu.VMEM((CH,), jnp.int32),
                pltpu.VMEM((CH,), jnp.float32),
                pltpu.VMEM((640,), jnp.float32),
            ],
        )
    return _SC_CACHE['deg'](dst)


NSET = 10
PREF = 4
NBUF = 4


def _scat_body(g2_hbm, src2_hbm, dst_hbm, acc_hbm, acc_s, sidx_v, didx_v,
               rows_v, gsem, ssem, isem):
    c = lax.axis_index("c")
    s = lax.axis_index("s")
    nchunks = EDGES_W // CH

    bounce = rows_v.at[0]

    def init(j, _):
        k = j * NS + s

        @pl.when(k < N // CH)
        def _():
            r = k * CH
            pltpu.sync_copy(g2_hbm.at[pl.ds(c * N + r, CH)], bounce)
            pltpu.sync_copy(bounce, acc_s.at[pl.ds(r, CH)])

        return 0

    lax.fori_loop(0, (N // CH + NS - 1) // NS, init, 0)
    plsc.subcore_barrier()

    ebase = s * EDGES_W

    def idx_load(chunk, q):
        off = ebase + chunk * CH
        pltpu.async_copy(src2_hbm.at[pl.ds(c * E + off, CH)],
                         sidx_v.at[q], isem.at[q])
        pltpu.async_copy(dst_hbm.at[pl.ds(off, CH)],
                         didx_v.at[q], isem.at[q])

    def idx_wait(chunk, q):
        off = ebase + chunk * CH
        pltpu.make_async_copy(src2_hbm.at[pl.ds(c * E + off, CH)],
                              sidx_v.at[q], isem.at[q]).wait()
        pltpu.make_async_copy(dst_hbm.at[pl.ds(off, CH)],
                              didx_v.at[q], isem.at[q]).wait()

    def gather_start(q, b):
        pltpu.async_copy(g2_hbm.at[sidx_v.at[q]], rows_v.at[b], gsem.at[b])

    def gather_wait(q, b):
        pltpu.make_async_copy(g2_hbm.at[sidx_v.at[q]], rows_v.at[b],
                              gsem.at[b]).wait()

    def scat_start(q, b):
        pltpu.async_copy(rows_v.at[b], acc_s.at[didx_v.at[q]], ssem.at[b],
                         add=True)

    def scat_wait(q, b):
        pltpu.make_async_copy(rows_v.at[b], acc_s.at[didx_v.at[q]],
                              ssem.at[b]).wait()

    for q in range(PREF):
        idx_load(q, q)
    idx_wait(0, 0)
    gather_start(0, 0)
    idx_wait(1, 1)
    gather_start(1, 1)

    def ring(t, _):
        for u in range(NSET):
            j = t * NSET + u
            bj = lax.rem(j, NBUF)
            b2 = lax.rem(j + 2, NBUF)
            if u >= 2:
                scat_wait((u - 2) % NSET, b2)
            else:
                @pl.when(j >= 2)
                def _():
                    scat_wait((u - 2) % NSET, b2)
            if u < NSET - 2:
                idx_wait(j + 2, u + 2)
                gather_start(u + 2, b2)
            else:
                @pl.when(j + 2 < nchunks)
                def _():
                    idx_wait(j + 2, (u + 2) % NSET)
                    gather_start((u + 2) % NSET, b2)
            gather_wait(u, bj)
            scat_start(u, bj)
            if u < NSET - PREF:
                idx_load(j + PREF, u + PREF)
            else:
                @pl.when(j + PREF < nchunks)
                def _():
                    idx_load(j + PREF, (u + PREF) % NSET)

        return 0

    lax.fori_loop(0, nchunks // NSET, ring, 0)

    last = nchunks - 1
    scat_wait((last - 1) % NSET, (last - 1) % NBUF)
    scat_wait(last % NSET, last % NBUF)
    plsc.subcore_barrier()

    def wb(j, _):
        k = j * NS + s

        @pl.when(k < N // CH)
        def _():
            r = k * CH
            pltpu.sync_copy(acc_s.at[pl.ds(r, CH)], bounce)
            pltpu.sync_copy(bounce, acc_hbm.at[pl.ds(c * N + r, CH)])

        return 0

    lax.fori_loop(0, (N // CH + NS - 1) // NS, wb, 0)


def _edge_scatter(g, src2, dst):
    if 'scat' not in _SC_CACHE:
        _SC_CACHE['scat'] = pl.kernel(
            _scat_body,
            out_type=jax.ShapeDtypeStruct((NC * N, HH), jnp.float32),
            mesh=_sc_mesh(),
            scratch_types=[
                pltpu.VMEM_SHARED((N, HH), jnp.float32),
                pltpu.VMEM((NSET, CH), jnp.int32),
                pltpu.VMEM((NSET, CH), jnp.int32),
                pltpu.VMEM((NBUF, CH, HH), jnp.float32),
                pltpu.SemaphoreType.DMA((NBUF,)),
                pltpu.SemaphoreType.DMA((NBUF,)),
                pltpu.SemaphoreType.DMA((NSET,)),
            ],
        )
    g2 = g.reshape(NC * N, HH)
    acc2 = _SC_CACHE['scat'](g2, src2, dst)
    return acc2.reshape(NC, N, HH)


def _dinv_of(degp_ref):
    deg = degp_ref[0] + degp_ref[1] + 1.0
    return lax.rsqrt(jnp.maximum(deg, 1.0))


def _mm0_body(x_ref, w_ref, degp_ref, o_ref):
    dinv = _dinv_of(degp_ref)
    h = jnp.dot(x_ref[...], w_ref[...], preferred_element_type=jnp.float32)
    o_ref[0] = h * dinv


def _mm0(x, w, degp3):
    return pl.pallas_call(
        _mm0_body,
        grid=(NB, 2),
        in_specs=[
            pl.BlockSpec((BN_BLOCK, D), lambda i, c: (i, 0)),
            pl.BlockSpec((D, HH), lambda i, c: (0, c)),
            pl.BlockSpec((2, BN_BLOCK, 1), lambda i, c: (0, i, 0)),
        ],
        out_specs=pl.BlockSpec((1, BN_BLOCK, HH), lambda i, c: (c, i, 0)),
        out_shape=jax.ShapeDtypeStruct((2, N, HH), jnp.float32),
    )(x, w, degp3)


def _mm_body(s0_ref, s1_ref, w0_ref, w1_ref, degp_ref, o_ref):
    dinv = _dinv_of(degp_ref)
    h = jnp.dot(s0_ref[0], w0_ref[0], preferred_element_type=jnp.float32)
    h = h + jnp.dot(s1_ref[0], w1_ref[0], preferred_element_type=jnp.float32)
    o_ref[0] = h * dinv


def _mm(s, w2, degp3):
    return pl.pallas_call(
        _mm_body,
        grid=(NB, 2),
        in_specs=[
            pl.BlockSpec((1, BN_BLOCK, HH), lambda i, c: (0, i, 0)),
            pl.BlockSpec((1, BN_BLOCK, HH), lambda i, c: (1, i, 0)),
            pl.BlockSpec((1, HH, HH), lambda i, c: (0, 0, c)),
            pl.BlockSpec((1, HH, HH), lambda i, c: (1, 0, c)),
            pl.BlockSpec((2, BN_BLOCK, 1), lambda i, c: (0, i, 0)),
        ],
        out_specs=pl.BlockSpec((1, BN_BLOCK, HH), lambda i, c: (c, i, 0)),
        out_shape=jax.ShapeDtypeStruct((2, N, HH), jnp.float32),
    )(s, s, w2, w2, degp3)


def _combine_body(acc_ref, degp_ref, b_ref, bng_ref, bnb_ref, o_ref):
    dinv = _dinv_of(degp_ref)
    t = acc_ref[0] * dinv + b_ref[0]
    t = t * (BNSCALE * bng_ref[0]) + bnb_ref[0]
    o_ref[0] = _relu(t)


def _combine_res_body(acc_ref, degp_ref, b_ref, bng_ref, bnb_ref, sprev_ref,
                      o_ref):
    dinv = _dinv_of(degp_ref)
    t = acc_ref[0] * dinv + b_ref[0]
    t = t * (BNSCALE * bng_ref[0]) + bnb_ref[0]
    o_ref[0] = _relu(t) + sprev_ref[0]


def _combine(acc, degp3, b2, bng2, bnb2, sprev=None):
    specs = [
        pl.BlockSpec((1, BN_BLOCK, HH), lambda i, c: (c, i, 0)),
        pl.BlockSpec((2, BN_BLOCK, 1), lambda i, c: (0, i, 0)),
        pl.BlockSpec((1, 1, HH), lambda i, c: (c, 0, 0)),
        pl.BlockSpec((1, 1, HH), lambda i, c: (c, 0, 0)),
        pl.BlockSpec((1, 1, HH), lambda i, c: (c, 0, 0)),
    ]
    args = [acc, degp3, b2, bng2, bnb2]
    body = _combine_body
    if sprev is not None:
        specs.append(pl.BlockSpec((1, BN_BLOCK, HH), lambda i, c: (c, i, 0)))
        args.append(sprev)
        body = _combine_res_body
    return pl.pallas_call(
        body,
        grid=(NB, 2),
        in_specs=specs,
        out_specs=pl.BlockSpec((1, BN_BLOCK, HH), lambda i, c: (c, i, 0)),
        out_shape=jax.ShapeDtypeStruct((2, N, HH), jnp.float32),
    )(*args)


def _pool_body(s0_ref, s1_ref, x_ref, b_ref, o_gx_ref, o_x_ref, o_cnt_ref):
    i = pl.program_id(0)

    @pl.when(i == 0)
    def _():
        o_gx_ref[...] = jnp.zeros_like(o_gx_ref)
        o_x_ref[...] = jnp.zeros_like(o_x_ref)
        o_cnt_ref[...] = jnp.zeros_like(o_cnt_ref)

    b = b_ref[0]
    gid = lax.broadcasted_iota(jnp.int32, (G, BN_BLOCK), 0)
    m = (gid == b).astype(jnp.float32)
    o_gx_ref[0] += jnp.dot(m, s0_ref[0], preferred_element_type=jnp.float32)
    o_gx_ref[1] += jnp.dot(m, s1_ref[0], preferred_element_type=jnp.float32)
    o_x_ref[...] += jnp.dot(m, x_ref[...], preferred_element_type=jnp.float32)
    o_cnt_ref[...] += jnp.broadcast_to(
        jnp.sum(m, axis=1, keepdims=True), (G, HH))


def _pool(s, x, batch3):
    return pl.pallas_call(
        _pool_body,
        grid=(NB,),
        in_specs=[
            pl.BlockSpec((1, BN_BLOCK, HH), lambda i: (0, i, 0)),
            pl.BlockSpec((1, BN_BLOCK, HH), lambda i: (1, i, 0)),
            pl.BlockSpec((BN_BLOCK, D), lambda i: (i, 0)),
            pl.BlockSpec((1, 1, BN_BLOCK), lambda i: (i, 0, 0)),
        ],
        out_specs=[
            pl.BlockSpec((2, G, HH), lambda i: (0, 0, 0)),
            pl.BlockSpec((G, D), lambda i: (0, 0)),
            pl.BlockSpec((G, HH), lambda i: (0, 0)),
        ],
        out_shape=[
            jax.ShapeDtypeStruct((2, G, HH), jnp.float32),
            jax.ShapeDtypeStruct((G, D), jnp.float32),
            jax.ShapeDtypeStruct((G, HH), jnp.float32),
        ],
    )(s, s, x, batch3)


def _head_body(sgx_ref, sx_ref, cnt_ref, dW1_ref, db1_ref, dg1_ref, dbb1_ref,
               dW2_ref, db2_ref, dg2_ref, dbb2_ref, fW1_ref, fb1_ref, fg1_ref,
               fbb1_ref, fW2_ref, fb2_ref, fg2_ref, fbb2_ref, fW3_ref,
               fb3_ref, o_ref):
    cnt = jnp.maximum(cnt_ref[...], 1.0)
    gr0 = sgx_ref[0] / cnt
    gr1 = sgx_ref[1] / cnt
    nr = sx_ref[...] / cnt[:, :D]

    def dot(a, b):
        return jnp.dot(a, b, preferred_element_type=jnp.float32)

    d1 = _relu((dot(nr, dW1_ref[...]) + db1_ref[...]) * (BNSCALE * dg1_ref[...])
               + dbb1_ref[...])
    d2 = _relu((dot(d1, dW2_ref[...]) + db2_ref[...]) * (BNSCALE * dg2_ref[...])
               + dbb2_ref[...])
    y1 = (dot(gr0, fW1_ref[0:HH]) + dot(gr1, fW1_ref[HH:2 * HH])
          + dot(d2, fW1_ref[2 * HH:4 * HH]) + fb1_ref[...])
    y1 = _relu(y1 * (BNSCALE * fg1_ref[...]) + fbb1_ref[...])
    y2 = _relu((dot(y1, fW2_ref[...]) + fb2_ref[...]) * (BNSCALE * fg2_ref[...])
               + fbb2_ref[...])
    o_ref[...] = dot(y2, fW3_ref[...]) + fb3_ref[...]


def _head(sgx, sx, cnt, p):
    r = lambda v: v.reshape(1, -1)
    args = [sgx, sx, cnt,
            p['d_W1'], r(p['d_b1']), r(p['d_bn_g1']), r(p['d_bn_b1']),
            p['d_W2'], r(p['d_b2']), r(p['d_bn_g2']), r(p['d_bn_b2']),
            p['f_W1'], r(p['f_b1']), r(p['f_bn_g1']), r(p['f_bn_b1']),
            p['f_W2'], r(p['f_b2']), r(p['f_bn_g2']), r(p['f_bn_b2']),
            p['f_W3'], r(p['f_b3'])]
    return pl.pallas_call(
        _head_body,
        out_shape=jax.ShapeDtypeStruct((G, 1), jnp.float32),
    )(*args)


def kernel(x, edge_index, batch, params):
    src = edge_index[0]
    dst = edge_index[1]
    src2 = jnp.concatenate([src, src + N])
    degp = _sc_degree(dst).reshape(NC, PADN)
    degp3 = degp[:, :N].reshape(2, N, 1)
    batch3 = batch.reshape(NB, 1, BN_BLOCK)

    g = _mm0(x, params['gcn_W_0'], degp3)
    acc = _edge_scatter(g, src2, dst)
    s = _combine(acc, degp3,
                 params['gcn_b_0'].reshape(2, 1, HH),
                 params['gcn_bn_g_0'].reshape(2, 1, HH),
                 params['gcn_bn_b_0'].reshape(2, 1, HH))
    for i in range(1, L):
        w2 = params[f'gcn_W_{i}'].reshape(2, HH, H)
        g = _mm(s, w2, degp3)
        acc = _edge_scatter(g, src2, dst)
        s = _combine(acc, degp3,
                     params[f'gcn_b_{i}'].reshape(2, 1, HH),
                     params[f'gcn_bn_g_{i}'].reshape(2, 1, HH),
                     params[f'gcn_bn_b_{i}'].reshape(2, 1, HH),
                     sprev=s)

    sgx, sx, cnt = _pool(s, x, batch3)
    return _head(sgx, sx, cnt, params)
